# baseline (speedup 1.0000x reference)
import functools

import jax
import jax.numpy as jnp
from jax import lax
from jax.experimental import pallas as pl
from jax.experimental.pallas import tpu as pltpu
from jax.experimental.pallas import tpu_sc as plsc

NUM_SEGMENTS = 10000
CHUNK = 128
CHUNK_C = 128
TRASH = 8
ROWS_BLK = 1000


def _leaky(h):
    return jnp.where(h > 0, h, 0.01 * h)


def _ln(h, w, b):
    mu = jnp.mean(h, axis=1, keepdims=True)
    var = jnp.mean((h - mu) ** 2, axis=1, keepdims=True)
    return (h - mu) * lax.rsqrt(var + 1e-5) * w + b


def _dotT(a, b):
    return lax.dot_general(a, b, (((1,), (1,)), ((), ())),
                           preferred_element_type=jnp.float32)


def _prologue_body(x_ref, wlin_ref, blin_ref, lnw_ref, lnb_ref, whg_ref,
                   wskip_ref, bskip_ref, xw0_ref, xw1_ref, skip_ref):
    h = _dotT(x_ref[...], wlin_ref[...]) + blin_ref[...]
    h = _ln(_leaky(h), lnw_ref[...], lnb_ref[...])
    xw = _dotT(h, whg_ref[...])
    xw0_ref[...] = xw[:, :128]
    xw1_ref[...] = xw[:, 128:]
    skip_ref[...] = _dotT(h, wskip_ref[...]) + bskip_ref[...]


def _scale_body(a0_ref, a1_ref, cnt_ref, o0_ref, o1_ref):
    c = cnt_ref[...][:, 0:1]
    inv = jnp.where(c == 0.0, 0.0, 1.0 / c)
    o0_ref[...] = a0_ref[...] * inv
    o1_ref[...] = a1_ref[...] * inv


def _epilogue_body(a0_ref, a1_ref, cnt_ref, bhg_ref, gnw_ref, gnb_ref,
                   skip_ref, out_ref):
    c = cnt_ref[...][:, 0:1]
    inv = jnp.where(c == 0.0, 0.0, 1.0 / c)
    conv = jnp.concatenate([a0_ref[...], a1_ref[...]], axis=1) * inv
    h = _leaky(conv + bhg_ref[...])
    out_ref[...] = _ln(h, gnw_ref[...], gnb_ref[...]) + skip_ref[...]


def _sub_rows(total):
    base = (total // 16) // 8 * 8
    last = total - 15 * base
    return base, last


def _n_chunks(n_edges):
    per_sub = n_edges // 16
    nch = -(-per_sub // CHUNK)
    return -(-nch // 6) * 6


def _n_chunks_c(n_edges):
    per_sub = n_edges // 16
    nch = -(-per_sub // CHUNK_C)
    return nch + (nch % 2)


def _each_slice(sid, rb, rl, fn):
    @pl.when(sid < 15)
    def _():
        fn(pl.multiple_of(sid * rb, 8), rb)

    @pl.when(sid == 15)
    def _():
        fn(15 * rb, rl)


def _make_sc_counts(n_nodes, n_edges):
    seg = NUM_SEGMENTS
    nch = _n_chunks_c(n_edges)
    half = nch // 2
    tbl = max(n_nodes, seg) + TRASH
    rb, rl = _sub_rows(max(n_nodes, seg))

    @functools.partial(
        pl.kernel,
        mesh=plsc.VectorSubcoreMesh(core_axis_name="c", subcore_axis_name="s"),
        out_type=[
            jax.ShapeDtypeStruct((n_nodes, 128), jnp.float32),
            jax.ShapeDtypeStruct((seg, 128), jnp.float32),
        ],
        scratch_types=[
            pltpu.VMEM_SHARED((tbl, 128), jnp.float32),
            pltpu.VMEM((CHUNK_C,), jnp.int32),
            pltpu.VMEM((CHUNK_C,), jnp.int32),
            pltpu.VMEM((CHUNK_C, 128), jnp.float32),
            pltpu.SemaphoreType.DMA,
            pltpu.SemaphoreType.DMA,
        ],
    )
    def counts(carr_n, carr_e, zeros_feat, ones_feat,
               dcnt, bcnt, cnt, idx_a, idx_b, ones_v, sem_sa, sem_sb):
        cid = lax.axis_index("c")
        sid = lax.axis_index("s")

        _each_slice(sid, rb, rl, lambda r0, sz: pltpu.sync_copy(
            zeros_feat.at[pl.ds(0, sz)], cnt.at[pl.ds(r0, sz)]))
        pltpu.sync_copy(ones_feat, ones_v)
        plsc.subcore_barrier()

        def run(carr):
            pltpu.sync_copy(carr.at[sid, 0], idx_a)
            pltpu.sync_copy(carr.at[sid, 1], idx_b)

            def body(i, carry):
                c0 = 2 * i
                pltpu.async_copy(ones_v, cnt.at[idx_a], sem_sa, add=True)
                pltpu.async_copy(ones_v, cnt.at[idx_b], sem_sb, add=True)

                @pl.when(i < half - 1)
                def _():
                    pltpu.make_async_copy(ones_v, cnt.at[idx_a],
                                          sem_sa).wait()
                    pltpu.sync_copy(carr.at[sid, c0 + 2], idx_a)
                    pltpu.make_async_copy(ones_v, cnt.at[idx_b],
                                          sem_sb).wait()
                    pltpu.sync_copy(carr.at[sid, c0 + 3], idx_b)
                return carry

            lax.fori_loop(0, half, body, 0)
            pltpu.make_async_copy(ones_v, cnt.at[idx_a], sem_sa).wait()
            pltpu.make_async_copy(ones_v, cnt.at[idx_b], sem_sb).wait()

        @pl.when(cid == 0)
        def _():
            run(carr_n)

        @pl.when(cid == 1)
        def _():
            run(carr_e)

        plsc.subcore_barrier()

        @pl.when(cid == 0)
        def _():
            _each_slice(sid, rb, rl, lambda r0, sz: pltpu.sync_copy(
                cnt.at[pl.ds(r0, sz)], dcnt.at[pl.ds(r0, sz)]))

        @pl.when(cid == 1)
        def _():
            _each_slice(sid, rb, rl, lambda r0, sz: pltpu.sync_copy(
                cnt.at[pl.ds(r0, sz)], bcnt.at[pl.ds(r0, sz)]))

    return counts


def _make_sc_stage(n_rows_out, n_edges):
    nch = _n_chunks(n_edges)
    q_iters = nch // 6
    rb, rl = _sub_rows(n_rows_out)

    @functools.partial(
        pl.kernel,
        mesh=plsc.VectorSubcoreMesh(core_axis_name="c", subcore_axis_name="s"),
        out_type=[
            jax.ShapeDtypeStruct((n_rows_out, 128), jnp.float32),
            jax.ShapeDtypeStruct((n_rows_out, 128), jnp.float32),
        ],
        scratch_types=[
            pltpu.VMEM_SHARED((n_rows_out + TRASH, 128), jnp.float32),
            pltpu.VMEM((3, 2, CHUNK), jnp.int32),
            pltpu.VMEM((3, 2, CHUNK), jnp.int32),
            pltpu.VMEM((CHUNK, 128), jnp.float32),
            pltpu.VMEM((CHUNK, 128), jnp.float32),
            pltpu.VMEM((CHUNK, 128), jnp.float32),
            pltpu.SemaphoreType.DMA,
            pltpu.SemaphoreType.DMA,
            pltpu.SemaphoreType.DMA,
            pltpu.SemaphoreType.DMA,
            pltpu.SemaphoreType.DMA,
            pltpu.SemaphoreType.DMA,
            pltpu.SemaphoreType.DMA,
            pltpu.SemaphoreType.DMA,
        ],
    )
    def stage(src0, src1, comb, zeros_feat,
              agg0, agg1, acc, x0, x1, rows_a, rows_b, rows_c,
              sem_ga, sem_gb, sem_gc, sem_sa, sem_sb, sem_sc,
              sem_x0, sem_x1):
        cid = lax.axis_index("c")
        sid = lax.axis_index("s")

        _each_slice(sid, rb, rl, lambda r0, sz: pltpu.sync_copy(
            zeros_feat.at[pl.ds(0, sz)], acc.at[pl.ds(r0, sz)]))
        plsc.subcore_barrier()

        rows = (rows_a, rows_b, rows_c)
        gsems = (sem_ga, sem_gb, sem_gc)
        ssems = (sem_sa, sem_sb, sem_sc)
        xs = (x0, x1)
        xsems = (sem_x0, sem_x1)

        def run(src):
            def gath(xc, jj, b):
                pltpu.async_copy(src.at[xc.at[jj, 0]], rows[b], gsems[b])

            def wait_gath(xc, jj, b):
                pltpu.make_async_copy(src.at[xc.at[jj, 0]], rows[b],
                                      gsems[b]).wait()

            def scat(xc, jj, b):
                pltpu.async_copy(rows[b], acc.at[xc.at[jj, 1]], ssems[b],
                                 add=True)

            def wait_scat(xc, jj, b):
                pltpu.make_async_copy(rows[b], acc.at[xc.at[jj, 1]],
                                      ssems[b]).wait()

            pltpu.sync_copy(comb.at[sid, pl.ds(0, 3)], x0)
            gath(x0, 0, 0)
            gath(x0, 1, 1)

            def body(i, carry):
                c6 = 6 * i
                for j in range(6):
                    c = c6 + j
                    b = j % 3
                    slab, jj = (j // 3), j % 3
                    xc = xs[slab]
                    pslab, pjj = ((j - 1) % 6) // 3, (j - 1) % 3
                    nslab, njj = ((j + 2) % 6) // 3, (j + 2) % 3

                    wait_gath(xc, jj, b)
                    scat(xc, jj, b)
                    if j == 0:
                        @pl.when(i > 0)
                        def _():
                            wait_scat(xs[pslab], pjj, (j - 1) % 3)
                    else:
                        wait_scat(xs[pslab], pjj, (j - 1) % 3)
                    if j == 0:
                        pltpu.async_copy(comb.at[sid, pl.ds(c6 + 3, 3)],
                                         x1, sem_x1)
                    if j == 3:
                        @pl.when(c <= nch - 6)
                        def _():
                            pltpu.async_copy(comb.at[sid, pl.ds(c6 + 6, 3)],
                                             x0, sem_x0)
                    if j == 1:
                        pltpu.make_async_copy(comb.at[sid, pl.ds(0, 3)],
                                              x1, sem_x1).wait()
                    if j == 4:
                        @pl.when(c <= nch - 5)
                        def _():
                            pltpu.make_async_copy(comb.at[sid, pl.ds(0, 3)],
                                                  x0, sem_x0).wait()

                    @pl.when(c < nch - 2)
                    def _():
                        gath(xs[nslab], njj, (j + 2) % 3)
                return carry

            lax.fori_loop(0, q_iters, body, 0)
            wait_scat(xs[1], 2, 2)

        @pl.when(cid == 0)
        def _():
            run(src0)

        @pl.when(cid == 1)
        def _():
            run(src1)

        plsc.subcore_barrier()

        @pl.when(cid == 0)
        def _():
            _each_slice(sid, rb, rl, lambda r0, sz: pltpu.sync_copy(
                acc.at[pl.ds(r0, sz)], agg0.at[pl.ds(r0, sz)]))

        @pl.when(cid == 1)
        def _():
            _each_slice(sid, rb, rl, lambda r0, sz: pltpu.sync_copy(
                acc.at[pl.ds(r0, sz)], agg1.at[pl.ds(r0, sz)]))

    return stage


def _pad_pairs(gidx, sidx, n_edges, trash_row):
    nch = _n_chunks(n_edges)
    per_sub = n_edges // 16
    pad = nch * CHUNK - per_sub
    g = jnp.pad(gidx.reshape(16, per_sub), ((0, 0), (0, pad)),
                constant_values=0).reshape(16, nch, CHUNK)
    s = jnp.pad(sidx.reshape(16, per_sub), ((0, 0), (0, pad)),
                constant_values=trash_row).reshape(16, nch, CHUNK)
    return jnp.stack([g, s], axis=2)


def _pad_counts(sidx, n_edges, trash_row):
    nch = _n_chunks_c(n_edges)
    per_sub = n_edges // 16
    pad = nch * CHUNK_C - per_sub
    return jnp.pad(sidx.reshape(16, per_sub), ((0, 0), (0, pad)),
                   constant_values=trash_row).reshape(16, nch, CHUNK_C)


def kernel(x, edge_index, W_lin, b_lin, ln_w, ln_b, W_hg, b_hg, gn_w, gn_b,
           W_skip, b_skip):
    n, in_c = x.shape
    hid = W_lin.shape[0]
    e = edge_index.shape[1]
    seg = NUM_SEGMENTS
    nidx = edge_index[0]
    eidx = edge_index[1]
    comb1 = _pad_pairs(nidx, eidx, e, seg)
    comb2 = _pad_pairs(eidx, nidx, e, n)
    carr_n = _pad_counts(nidx, e, n)
    carr_e = _pad_counts(eidx, e, seg)

    grid = n // ROWS_BLK
    full = lambda shp: pl.BlockSpec(shp, lambda i: (0,) * len(shp))
    row_blk = lambda w: pl.BlockSpec((ROWS_BLK, w), lambda i: (i, 0))

    xw0, xw1, skip = pl.pallas_call(
        _prologue_body,
        grid=(grid,),
        in_specs=[row_blk(in_c), full((hid, in_c)), full((1, hid)),
                  full((1, hid)), full((1, hid)), full((hid, hid)),
                  full((hid, hid)), full((1, hid))],
        out_specs=[row_blk(128), row_blk(128), row_blk(hid)],
        out_shape=[jax.ShapeDtypeStruct((n, 128), jnp.float32),
                   jax.ShapeDtypeStruct((n, 128), jnp.float32),
                   jax.ShapeDtypeStruct((n, hid), jnp.float32)],
    )(x, W_lin, b_lin.reshape(1, hid), ln_w.reshape(1, hid),
      ln_b.reshape(1, hid), W_hg, W_skip, b_skip.reshape(1, hid))

    zrows = max(_sub_rows(seg)[1], _sub_rows(n)[1])
    zeros_feat = jnp.zeros((zrows, 128), jnp.float32)
    ones_feat = jnp.ones((CHUNK_C, 128), jnp.float32)

    dcnt, bcnt = _make_sc_counts(n, e)(carr_n, carr_e, zeros_feat, ones_feat)
    agg0, agg1 = _make_sc_stage(seg, e)(xw0, xw1, comb1, zeros_feat)

    out1_0, out1_1 = pl.pallas_call(
        _scale_body,
        grid=(seg // ROWS_BLK,),
        in_specs=[row_blk(128), row_blk(128), row_blk(128)],
        out_specs=[row_blk(128), row_blk(128)],
        out_shape=[jax.ShapeDtypeStruct((seg, 128), jnp.float32),
                   jax.ShapeDtypeStruct((seg, 128), jnp.float32)],
    )(agg0, agg1, bcnt)

    agg2_0, agg2_1 = _make_sc_stage(n, e)(out1_0, out1_1, comb2, zeros_feat)

    out = pl.pallas_call(
        _epilogue_body,
        grid=(grid,),
        in_specs=[row_blk(128), row_blk(128), row_blk(128), full((1, hid)),
                  full((1, hid)), full((1, hid)), row_blk(hid)],
        out_specs=row_blk(hid),
        out_shape=jax.ShapeDtypeStruct((n, hid), jnp.float32),
    )(agg2_0, agg2_1, dcnt, b_hg.reshape(1, hid), gn_w.reshape(1, hid),
      gn_b.reshape(1, hid), skip)

    return out


# speedup vs baseline: 1.3019x; 1.3019x over previous
import functools

import jax
import jax.numpy as jnp
from jax import lax
from jax.experimental import pallas as pl
from jax.experimental.pallas import tpu as pltpu
from jax.experimental.pallas import tpu_sc as plsc

NUM_SEGMENTS = 10000
CHUNK = 128
TRASH = 8
ROWS_BLK = 1000


def _leaky(h):
    return jnp.where(h > 0, h, 0.01 * h)


def _ln(h, w, b):
    mu = jnp.mean(h, axis=1, keepdims=True)
    var = jnp.mean((h - mu) ** 2, axis=1, keepdims=True)
    return (h - mu) * lax.rsqrt(var + 1e-5) * w + b


def _dotT(a, b):
    return lax.dot_general(a, b, (((1,), (1,)), ((), ())),
                           preferred_element_type=jnp.float32)


def _prologue_body(x_ref, wlin_ref, blin_ref, lnw_ref, lnb_ref, whg_ref,
                   wskip_ref, bskip_ref, xw0_ref, xw1_ref, skip_ref):
    h = _dotT(x_ref[...], wlin_ref[...]) + blin_ref[...]
    h = _ln(_leaky(h), lnw_ref[...], lnb_ref[...])
    xw = _dotT(h, whg_ref[...])
    xw0_ref[...] = xw[:, :128]
    xw1_ref[...] = xw[:, 128:]
    skip_ref[...] = _dotT(h, wskip_ref[...]) + bskip_ref[...]


def _scale_body(a0_ref, a1_ref, cnt_ref, o0_ref, o1_ref):
    c = cnt_ref[...][:, 0:1]
    inv = jnp.where(c == 0.0, 0.0, 1.0 / c)
    o0_ref[...] = a0_ref[...] * inv
    o1_ref[...] = a1_ref[...] * inv


def _epilogue_body(a0_ref, a1_ref, cnt_ref, bhg_ref, gnw_ref, gnb_ref,
                   skip_ref, out_ref):
    c = cnt_ref[...][:, 0:1]
    inv = jnp.where(c == 0.0, 0.0, 1.0 / c)
    conv = jnp.concatenate([a0_ref[...], a1_ref[...]], axis=1) * inv
    h = _leaky(conv + bhg_ref[...])
    out_ref[...] = _ln(h, gnw_ref[...], gnb_ref[...]) + skip_ref[...]


def _sub_rows(total):
    base = (total // 16) // 8 * 8
    last = total - 15 * base
    return base, last


def _n_chunks(n_edges):
    per_sub = n_edges // 16
    nch = -(-per_sub // CHUNK)
    return nch + (nch % 2)


def _each_slice(sid, rb, rl, fn):
    @pl.when(sid < 15)
    def _():
        fn(pl.multiple_of(sid * rb, 8), rb)

    @pl.when(sid == 15)
    def _():
        fn(15 * rb, rl)


def _make_sc_counts(n_nodes, n_edges):
    seg = NUM_SEGMENTS
    nch = _n_chunks(n_edges)
    half = nch // 2
    tbl = max(n_nodes, seg) + TRASH
    rb, rl = _sub_rows(max(n_nodes, seg))

    @functools.partial(
        pl.kernel,
        mesh=plsc.VectorSubcoreMesh(core_axis_name="c", subcore_axis_name="s"),
        out_type=[
            jax.ShapeDtypeStruct((n_nodes, 128), jnp.float32),
            jax.ShapeDtypeStruct((seg, 128), jnp.float32),
        ],
        scratch_types=[
            pltpu.VMEM_SHARED((tbl, 128), jnp.float32),
            pltpu.VMEM((2, CHUNK), jnp.int32),
            pltpu.VMEM((2, CHUNK), jnp.int32),
            pltpu.VMEM((CHUNK, 128), jnp.float32),
            pltpu.SemaphoreType.DMA,
            pltpu.SemaphoreType.DMA,
        ],
    )
    def counts(comb1, comb2, zeros_feat, ones_feat,
               dcnt, bcnt, cnt, idx_a, idx_b, ones_v, sem_sa, sem_sb):
        cid = lax.axis_index("c")
        sid = lax.axis_index("s")

        _each_slice(sid, rb, rl, lambda r0, sz: pltpu.sync_copy(
            zeros_feat.at[pl.ds(0, sz)], cnt.at[pl.ds(r0, sz)]))
        pltpu.sync_copy(ones_feat, ones_v)
        plsc.subcore_barrier()

        def run(comb):
            pltpu.sync_copy(comb.at[sid, 0], idx_a)
            pltpu.sync_copy(comb.at[sid, 1], idx_b)

            def body(i, carry):
                c0 = 2 * i
                pltpu.async_copy(ones_v, cnt.at[idx_a.at[1]], sem_sa,
                                 add=True)
                pltpu.async_copy(ones_v, cnt.at[idx_b.at[1]], sem_sb,
                                 add=True)

                @pl.when(i < half - 1)
                def _():
                    pltpu.make_async_copy(ones_v, cnt.at[idx_a.at[1]],
                                          sem_sa).wait()
                    pltpu.sync_copy(comb.at[sid, c0 + 2], idx_a)
                    pltpu.make_async_copy(ones_v, cnt.at[idx_b.at[1]],
                                          sem_sb).wait()
                    pltpu.sync_copy(comb.at[sid, c0 + 3], idx_b)
                return carry

            lax.fori_loop(0, half, body, 0)
            pltpu.make_async_copy(ones_v, cnt.at[idx_a.at[1]], sem_sa).wait()
            pltpu.make_async_copy(ones_v, cnt.at[idx_b.at[1]], sem_sb).wait()

        @pl.when(cid == 0)
        def _():
            run(comb2)

        @pl.when(cid == 1)
        def _():
            run(comb1)

        plsc.subcore_barrier()

        @pl.when(cid == 0)
        def _():
            _each_slice(sid, rb, rl, lambda r0, sz: pltpu.sync_copy(
                cnt.at[pl.ds(r0, sz)], dcnt.at[pl.ds(r0, sz)]))

        @pl.when(cid == 1)
        def _():
            _each_slice(sid, rb, rl, lambda r0, sz: pltpu.sync_copy(
                cnt.at[pl.ds(r0, sz)], bcnt.at[pl.ds(r0, sz)]))

    return counts


def _make_sc_stage(n_rows_out, n_edges):
    nch = _n_chunks(n_edges)
    half = nch // 2
    rb, rl = _sub_rows(n_rows_out)

    @functools.partial(
        pl.kernel,
        mesh=plsc.VectorSubcoreMesh(core_axis_name="c", subcore_axis_name="s"),
        out_type=[
            jax.ShapeDtypeStruct((n_rows_out, 128), jnp.float32),
            jax.ShapeDtypeStruct((n_rows_out, 128), jnp.float32),
        ],
        scratch_types=[
            pltpu.VMEM_SHARED((n_rows_out + TRASH, 128), jnp.float32),
            pltpu.VMEM((2, 2, CHUNK), jnp.int32),
            pltpu.VMEM((CHUNK, 128), jnp.float32),
            pltpu.VMEM((CHUNK, 128), jnp.float32),
            pltpu.SemaphoreType.DMA,
            pltpu.SemaphoreType.DMA,
            pltpu.SemaphoreType.DMA,
            pltpu.SemaphoreType.DMA,
        ],
    )
    def stage(src0, src1, comb, zeros_feat,
              agg0, agg1, acc, idx_ab, rows_a, rows_b,
              sem_ga, sem_gb, sem_sa, sem_sb):
        cid = lax.axis_index("c")
        sid = lax.axis_index("s")

        _each_slice(sid, rb, rl, lambda r0, sz: pltpu.sync_copy(
            zeros_feat.at[pl.ds(0, sz)], acc.at[pl.ds(r0, sz)]))
        plsc.subcore_barrier()

        def run(src):
            idx_a = idx_ab.at[0]
            idx_b = idx_ab.at[1]
            pltpu.sync_copy(comb.at[sid, pl.ds(0, 2)], idx_ab)
            pltpu.async_copy(src.at[idx_a.at[0]], rows_a, sem_ga)
            pltpu.async_copy(src.at[idx_b.at[0]], rows_b, sem_gb)

            def body(i, carry):
                c0 = 2 * i
                pltpu.make_async_copy(src.at[idx_a.at[0]], rows_a,
                                      sem_ga).wait()
                pltpu.async_copy(rows_a, acc.at[idx_a.at[1]], sem_sa,
                                 add=True)
                pltpu.make_async_copy(src.at[idx_b.at[0]], rows_b,
                                      sem_gb).wait()
                pltpu.async_copy(rows_b, acc.at[idx_b.at[1]], sem_sb,
                                 add=True)

                @pl.when(i < half - 1)
                def _():
                    pltpu.make_async_copy(rows_a, acc.at[idx_a.at[1]],
                                          sem_sa).wait()
                    pltpu.make_async_copy(rows_b, acc.at[idx_b.at[1]],
                                          sem_sb).wait()
                    pltpu.sync_copy(comb.at[sid, pl.ds(c0 + 2, 2)], idx_ab)
                    pltpu.async_copy(src.at[idx_a.at[0]], rows_a, sem_ga)
                    pltpu.async_copy(src.at[idx_b.at[0]], rows_b, sem_gb)
                return carry

            lax.fori_loop(0, half, body, 0)
            pltpu.make_async_copy(rows_a, acc.at[idx_a.at[1]], sem_sa).wait()
            pltpu.make_async_copy(rows_b, acc.at[idx_b.at[1]], sem_sb).wait()

        @pl.when(cid == 0)
        def _():
            run(src0)

        @pl.when(cid == 1)
        def _():
            run(src1)

        plsc.subcore_barrier()

        @pl.when(cid == 0)
        def _():
            _each_slice(sid, rb, rl, lambda r0, sz: pltpu.sync_copy(
                acc.at[pl.ds(r0, sz)], agg0.at[pl.ds(r0, sz)]))

        @pl.when(cid == 1)
        def _():
            _each_slice(sid, rb, rl, lambda r0, sz: pltpu.sync_copy(
                acc.at[pl.ds(r0, sz)], agg1.at[pl.ds(r0, sz)]))

    return stage


def _pad_pairs(gidx, sidx, n_edges, trash_row):
    nch = _n_chunks(n_edges)
    per_sub = n_edges // 16
    pad = nch * CHUNK - per_sub
    g = jnp.pad(gidx.reshape(16, per_sub), ((0, 0), (0, pad)),
                constant_values=0).reshape(16, nch, CHUNK)
    s = jnp.pad(sidx.reshape(16, per_sub), ((0, 0), (0, pad)),
                constant_values=trash_row).reshape(16, nch, CHUNK)
    return jnp.stack([g, s], axis=2)


def kernel(x, edge_index, W_lin, b_lin, ln_w, ln_b, W_hg, b_hg, gn_w, gn_b,
           W_skip, b_skip):
    n, in_c = x.shape
    hid = W_lin.shape[0]
    e = edge_index.shape[1]
    seg = NUM_SEGMENTS
    nidx = edge_index[0]
    eidx = edge_index[1]
    comb1 = _pad_pairs(nidx, eidx, e, seg)
    comb2 = _pad_pairs(eidx, nidx, e, n)

    grid = n // ROWS_BLK
    full = lambda shp: pl.BlockSpec(shp, lambda i: (0,) * len(shp))
    row_blk = lambda w: pl.BlockSpec((ROWS_BLK, w), lambda i: (i, 0))

    xw0, xw1, skip = pl.pallas_call(
        _prologue_body,
        grid=(grid,),
        in_specs=[row_blk(in_c), full((hid, in_c)), full((1, hid)),
                  full((1, hid)), full((1, hid)), full((hid, hid)),
                  full((hid, hid)), full((1, hid))],
        out_specs=[row_blk(128), row_blk(128), row_blk(hid)],
        out_shape=[jax.ShapeDtypeStruct((n, 128), jnp.float32),
                   jax.ShapeDtypeStruct((n, 128), jnp.float32),
                   jax.ShapeDtypeStruct((n, hid), jnp.float32)],
    )(x, W_lin, b_lin.reshape(1, hid), ln_w.reshape(1, hid),
      ln_b.reshape(1, hid), W_hg, W_skip, b_skip.reshape(1, hid))

    zrows = max(_sub_rows(seg)[1], _sub_rows(n)[1])
    zeros_feat = jnp.zeros((zrows, 128), jnp.float32)
    ones_feat = jnp.ones((CHUNK, 128), jnp.float32)

    dcnt, bcnt = _make_sc_counts(n, e)(comb1, comb2, zeros_feat, ones_feat)
    agg0, agg1 = _make_sc_stage(seg, e)(xw0, xw1, comb1, zeros_feat)

    out1_0, out1_1 = pl.pallas_call(
        _scale_body,
        grid=(seg // ROWS_BLK,),
        in_specs=[row_blk(128), row_blk(128), row_blk(128)],
        out_specs=[row_blk(128), row_blk(128)],
        out_shape=[jax.ShapeDtypeStruct((seg, 128), jnp.float32),
                   jax.ShapeDtypeStruct((seg, 128), jnp.float32)],
    )(agg0, agg1, bcnt)

    agg2_0, agg2_1 = _make_sc_stage(n, e)(out1_0, out1_1, comb2, zeros_feat)

    out = pl.pallas_call(
        _epilogue_body,
        grid=(grid,),
        in_specs=[row_blk(128), row_blk(128), row_blk(128), full((1, hid)),
                  full((1, hid)), full((1, hid)), row_blk(hid)],
        out_specs=row_blk(hid),
        out_shape=jax.ShapeDtypeStruct((n, hid), jnp.float32),
    )(agg2_0, agg2_1, dcnt, b_hg.reshape(1, hid), gn_w.reshape(1, hid),
      gn_b.reshape(1, hid), skip)

    return out


# speedup vs baseline: 1.4706x; 1.1296x over previous
import functools

import jax
import jax.numpy as jnp
from jax import lax
from jax.experimental import pallas as pl
from jax.experimental.pallas import tpu as pltpu
from jax.experimental.pallas import tpu_sc as plsc

NUM_SEGMENTS = 10000
CHUNK = 128
TRASH = 8
ROWS_BLK = 1000


def _leaky(h):
    return jnp.where(h > 0, h, 0.01 * h)


def _ln(h, w, b):
    mu = jnp.mean(h, axis=1, keepdims=True)
    var = jnp.mean((h - mu) ** 2, axis=1, keepdims=True)
    return (h - mu) * lax.rsqrt(var + 1e-5) * w + b


def _dotT(a, b):
    return lax.dot_general(a, b, (((1,), (1,)), ((), ())),
                           preferred_element_type=jnp.float32)


def _prologue_body(x_ref, wlin_ref, blin_ref, lnw_ref, lnb_ref, whg_ref,
                   wskip_ref, bskip_ref, xw0_ref, xw1_ref, skip_ref):
    h = _dotT(x_ref[...], wlin_ref[...]) + blin_ref[...]
    h = _ln(_leaky(h), lnw_ref[...], lnb_ref[...])
    xw = _dotT(h, whg_ref[...])
    xw0_ref[...] = xw[:, :128]
    xw1_ref[...] = xw[:, 128:]
    skip_ref[...] = _dotT(h, wskip_ref[...]) + bskip_ref[...]


def _scale_body(a0_ref, a1_ref, cnt_ref, o0_ref, o1_ref):
    c = cnt_ref[...][:, 0:1]
    inv = jnp.where(c == 0.0, 0.0, 1.0 / c)
    o0_ref[...] = a0_ref[...] * inv
    o1_ref[...] = a1_ref[...] * inv


def _epilogue_body(a0_ref, a1_ref, cnt_ref, bhg_ref, gnw_ref, gnb_ref,
                   skip_ref, out_ref):
    c = cnt_ref[...][:, 0:1]
    inv = jnp.where(c == 0.0, 0.0, 1.0 / c)
    conv = jnp.concatenate([a0_ref[...], a1_ref[...]], axis=1) * inv
    h = _leaky(conv + bhg_ref[...])
    out_ref[...] = _ln(h, gnw_ref[...], gnb_ref[...]) + skip_ref[...]


def _sub_rows(total):
    base = (total // 16) // 8 * 8
    last = total - 15 * base
    return base, last


def _n_chunks(n_edges):
    per_sub = n_edges // 16
    nch = -(-per_sub // CHUNK)
    return nch + (nch % 2)


def _each_slice(sid, rb, rl, fn):
    @pl.when(sid < 15)
    def _():
        fn(pl.multiple_of(sid * rb, 8), rb)

    @pl.when(sid == 15)
    def _():
        fn(15 * rb, rl)


def _make_sc_counts(n_nodes, n_edges):
    seg = NUM_SEGMENTS
    nch = _n_chunks(n_edges)
    half = nch // 2
    tbl = max(n_nodes, seg) + TRASH
    rb, rl = _sub_rows(max(n_nodes, seg))

    @functools.partial(
        pl.kernel,
        mesh=plsc.VectorSubcoreMesh(core_axis_name="c", subcore_axis_name="s"),
        out_type=[
            jax.ShapeDtypeStruct((n_nodes, 128), jnp.float32),
            jax.ShapeDtypeStruct((seg, 128), jnp.float32),
        ],
        scratch_types=[
            pltpu.VMEM_SHARED((tbl, 128), jnp.float32),
            pltpu.VMEM((2, CHUNK), jnp.int32),
            pltpu.VMEM((2, CHUNK), jnp.int32),
            pltpu.VMEM((CHUNK, 128), jnp.float32),
            pltpu.SemaphoreType.DMA,
            pltpu.SemaphoreType.DMA,
        ],
    )
    def counts(comb1, comb2, zeros_feat, ones_feat,
               dcnt, bcnt, cnt, idx_a, idx_b, ones_v, sem_sa, sem_sb):
        cid = lax.axis_index("c")
        sid = lax.axis_index("s")

        _each_slice(sid, rb, rl, lambda r0, sz: pltpu.sync_copy(
            zeros_feat.at[pl.ds(0, sz)], cnt.at[pl.ds(r0, sz)]))
        pltpu.sync_copy(ones_feat, ones_v)
        plsc.subcore_barrier()

        def run(comb):
            pltpu.sync_copy(comb.at[sid, 0], idx_a)
            pltpu.sync_copy(comb.at[sid, 1], idx_b)

            def body(i, carry):
                c0 = 2 * i
                pltpu.async_copy(ones_v, cnt.at[idx_a.at[1]], sem_sa,
                                 add=True)
                pltpu.async_copy(ones_v, cnt.at[idx_b.at[1]], sem_sb,
                                 add=True)

                @pl.when(i < half - 1)
                def _():
                    pltpu.make_async_copy(ones_v, cnt.at[idx_a.at[1]],
                                          sem_sa).wait()
                    pltpu.sync_copy(comb.at[sid, c0 + 2], idx_a)
                    pltpu.make_async_copy(ones_v, cnt.at[idx_b.at[1]],
                                          sem_sb).wait()
                    pltpu.sync_copy(comb.at[sid, c0 + 3], idx_b)
                return carry

            lax.fori_loop(0, half, body, 0)
            pltpu.make_async_copy(ones_v, cnt.at[idx_a.at[1]], sem_sa).wait()
            pltpu.make_async_copy(ones_v, cnt.at[idx_b.at[1]], sem_sb).wait()

        @pl.when(cid == 0)
        def _():
            run(comb2)

        @pl.when(cid == 1)
        def _():
            run(comb1)

        plsc.subcore_barrier()

        @pl.when(cid == 0)
        def _():
            _each_slice(sid, rb, rl, lambda r0, sz: pltpu.sync_copy(
                cnt.at[pl.ds(r0, sz)], dcnt.at[pl.ds(r0, sz)]))

        @pl.when(cid == 1)
        def _():
            _each_slice(sid, rb, rl, lambda r0, sz: pltpu.sync_copy(
                cnt.at[pl.ds(r0, sz)], bcnt.at[pl.ds(r0, sz)]))

    return counts


def _make_sc_stage(n_rows_out, n_edges):
    nch = _n_chunks(n_edges)
    half = nch // 2
    rb, rl = _sub_rows(n_rows_out)

    @functools.partial(
        pl.kernel,
        mesh=plsc.VectorSubcoreMesh(core_axis_name="c", subcore_axis_name="s"),
        out_type=[
            jax.ShapeDtypeStruct((n_rows_out, 128), jnp.float32),
            jax.ShapeDtypeStruct((n_rows_out, 128), jnp.float32),
        ],
        scratch_types=[
            pltpu.VMEM_SHARED((n_rows_out + TRASH, 128), jnp.float32),
            pltpu.VMEM((2, CHUNK), jnp.int32),
            pltpu.VMEM((2, CHUNK), jnp.int32),
            pltpu.VMEM((CHUNK, 128), jnp.float32),
            pltpu.VMEM((CHUNK, 128), jnp.float32),
            pltpu.SemaphoreType.DMA,
            pltpu.SemaphoreType.DMA,
            pltpu.SemaphoreType.DMA,
            pltpu.SemaphoreType.DMA,
        ],
    )
    def stage(src0, src1, comb, zeros_feat,
              agg0, agg1, acc, idx_a, idx_b, rows_a, rows_b,
              sem_ga, sem_gb, sem_sa, sem_sb):
        cid = lax.axis_index("c")
        sid = lax.axis_index("s")

        _each_slice(sid, rb, rl, lambda r0, sz: pltpu.sync_copy(
            zeros_feat.at[pl.ds(0, sz)], acc.at[pl.ds(r0, sz)]))
        plsc.subcore_barrier()

        def run(src):
            pltpu.sync_copy(comb.at[sid, 0], idx_a)
            pltpu.sync_copy(comb.at[sid, 1], idx_b)
            pltpu.async_copy(src.at[idx_a.at[0]], rows_a, sem_ga)
            pltpu.async_copy(src.at[idx_b.at[0]], rows_b, sem_gb)

            def body(i, carry):
                c0 = 2 * i
                pltpu.make_async_copy(src.at[idx_a.at[0]], rows_a,
                                      sem_ga).wait()
                pltpu.async_copy(rows_a, acc.at[idx_a.at[1]], sem_sa,
                                 add=True)
                pltpu.make_async_copy(src.at[idx_b.at[0]], rows_b,
                                      sem_gb).wait()
                pltpu.async_copy(rows_b, acc.at[idx_b.at[1]], sem_sb,
                                 add=True)

                @pl.when(i < half - 1)
                def _():
                    pltpu.make_async_copy(rows_a, acc.at[idx_a.at[1]],
                                          sem_sa).wait()
                    pltpu.sync_copy(comb.at[sid, c0 + 2], idx_a)
                    pltpu.async_copy(src.at[idx_a.at[0]], rows_a, sem_ga)
                    pltpu.make_async_copy(rows_b, acc.at[idx_b.at[1]],
                                          sem_sb).wait()
                    pltpu.sync_copy(comb.at[sid, c0 + 3], idx_b)
                    pltpu.async_copy(src.at[idx_b.at[0]], rows_b, sem_gb)
                return carry

            lax.fori_loop(0, half, body, 0)
            pltpu.make_async_copy(rows_a, acc.at[idx_a.at[1]], sem_sa).wait()
            pltpu.make_async_copy(rows_b, acc.at[idx_b.at[1]], sem_sb).wait()

        @pl.when(cid == 0)
        def _():
            run(src0)

        @pl.when(cid == 1)
        def _():
            run(src1)

        plsc.subcore_barrier()

        @pl.when(cid == 0)
        def _():
            _each_slice(sid, rb, rl, lambda r0, sz: pltpu.sync_copy(
                acc.at[pl.ds(r0, sz)], agg0.at[pl.ds(r0, sz)]))

        @pl.when(cid == 1)
        def _():
            _each_slice(sid, rb, rl, lambda r0, sz: pltpu.sync_copy(
                acc.at[pl.ds(r0, sz)], agg1.at[pl.ds(r0, sz)]))

    return stage


def _pad_pairs(gidx, sidx, n_edges, trash_row):
    nch = _n_chunks(n_edges)
    per_sub = n_edges // 16
    pad = nch * CHUNK - per_sub
    g = jnp.pad(gidx.reshape(16, per_sub), ((0, 0), (0, pad)),
                constant_values=0).reshape(16, nch, CHUNK)
    s = jnp.pad(sidx.reshape(16, per_sub), ((0, 0), (0, pad)),
                constant_values=trash_row).reshape(16, nch, CHUNK)
    return jnp.stack([g, s], axis=2)


def kernel(x, edge_index, W_lin, b_lin, ln_w, ln_b, W_hg, b_hg, gn_w, gn_b,
           W_skip, b_skip):
    n, in_c = x.shape
    hid = W_lin.shape[0]
    e = edge_index.shape[1]
    seg = NUM_SEGMENTS
    nidx = edge_index[0]
    eidx = edge_index[1]
    comb1 = _pad_pairs(nidx, eidx, e, seg)
    comb2 = _pad_pairs(eidx, nidx, e, n)

    grid = n // ROWS_BLK
    full = lambda shp: pl.BlockSpec(shp, lambda i: (0,) * len(shp))
    row_blk = lambda w: pl.BlockSpec((ROWS_BLK, w), lambda i: (i, 0))

    xw0, xw1, skip = pl.pallas_call(
        _prologue_body,
        grid=(grid,),
        in_specs=[row_blk(in_c), full((hid, in_c)), full((1, hid)),
                  full((1, hid)), full((1, hid)), full((hid, hid)),
                  full((hid, hid)), full((1, hid))],
        out_specs=[row_blk(128), row_blk(128), row_blk(hid)],
        out_shape=[jax.ShapeDtypeStruct((n, 128), jnp.float32),
                   jax.ShapeDtypeStruct((n, 128), jnp.float32),
                   jax.ShapeDtypeStruct((n, hid), jnp.float32)],
    )(x, W_lin, b_lin.reshape(1, hid), ln_w.reshape(1, hid),
      ln_b.reshape(1, hid), W_hg, W_skip, b_skip.reshape(1, hid))

    zrows = max(_sub_rows(seg)[1], _sub_rows(n)[1])
    zeros_feat = jnp.zeros((zrows, 128), jnp.float32)
    ones_feat = jnp.ones((CHUNK, 128), jnp.float32)

    dcnt, bcnt = _make_sc_counts(n, e)(comb1, comb2, zeros_feat, ones_feat)
    agg0, agg1 = _make_sc_stage(seg, e)(xw0, xw1, comb1, zeros_feat)

    out1_0, out1_1 = pl.pallas_call(
        _scale_body,
        grid=(seg // ROWS_BLK,),
        in_specs=[row_blk(128), row_blk(128), row_blk(128)],
        out_specs=[row_blk(128), row_blk(128)],
        out_shape=[jax.ShapeDtypeStruct((seg, 128), jnp.float32),
                   jax.ShapeDtypeStruct((seg, 128), jnp.float32)],
    )(agg0, agg1, bcnt)

    agg2_0, agg2_1 = _make_sc_stage(n, e)(out1_0, out1_1, comb2, zeros_feat)

    out = pl.pallas_call(
        _epilogue_body,
        grid=(grid,),
        in_specs=[row_blk(128), row_blk(128), row_blk(128), full((1, hid)),
                  full((1, hid)), full((1, hid)), row_blk(hid)],
        out_specs=row_blk(hid),
        out_shape=jax.ShapeDtypeStruct((n, hid), jnp.float32),
    )(agg2_0, agg2_1, dcnt, b_hg.reshape(1, hid), gn_w.reshape(1, hid),
      gn_b.reshape(1, hid), skip)

    return out
